# baseline (speedup 1.0000x reference)
import functools

import jax
import jax.numpy as jnp
import numpy as np
from jax import lax
from jax.experimental import pallas as pl
from jax.experimental.pallas import tpu as pltpu
from jax.experimental.pallas import tpu_sc as plsc

SEQ_VEC_SHAPE = 32
MAX_SEQ_LENGTH = 200
VOCAB_ROWS = 100000
BATCH = 4096

_NC = 2
_NS = 16
_NW = _NC * _NS
_BBLK = BATCH // _NW
_TC = 4
_NCHUNK = MAX_SEQ_LENGTH // _TC
_SCALE = float(np.sqrt(float(SEQ_VEC_SHAPE)))
_D = SEQ_VEC_SHAPE


def _pos_table() -> np.ndarray:
    initial_positions = np.arange(MAX_SEQ_LENGTH)[:, np.newaxis]
    positions = np.repeat(initial_positions, SEQ_VEC_SHAPE, axis=1)
    angle_rads = positions * (1.0 / 1000.0)
    s = np.sin(angle_rads)[::2]
    c = 1.0 - np.cos(angle_rads)[1::2]
    return np.vstack([s, c]).astype(np.float32)


_POS = _pos_table()


@functools.partial(
    pl.kernel,
    mesh=plsc.VectorSubcoreMesh(core_axis_name="c", subcore_axis_name="s"),
    out_type=jax.ShapeDtypeStruct((MAX_SEQ_LENGTH, _D // 8, _NW, 8 * _BBLK),
                                  jnp.float32),
    scratch_types=[
        pltpu.VMEM((_TC, _BBLK), jnp.int32),
        pltpu.VMEM((_TC * _BBLK, _D), jnp.float32),
        pltpu.VMEM((_TC, _D // 8, 8 * _BBLK), jnp.float32),
        pltpu.VMEM((MAX_SEQ_LENGTH * _D,), jnp.float32),
        pltpu.SemaphoreType.DMA,
    ],
    compiler_params=pltpu.CompilerParams(
        use_tc_tiling_on_sc=False, needs_layout_passes=False
    ),
)
def _sc_embed(idx_hbm, table_hbm, pos_hbm, out_hbm, idx_v, rows_v, out_v, pos_v, sem):
    wid = lax.axis_index("s") * _NC + lax.axis_index("c")
    lanes = lax.iota(jnp.int32, 16)

    pltpu.sync_copy(pos_hbm, pos_v)

    def chunk_body(c, carry):
        t0 = c * _TC
        pltpu.sync_copy(
            idx_hbm.at[pl.ds(t0, _TC), pl.ds(wid * _BBLK, _BBLK)], idx_v
        )
        copies = [
            pltpu.async_copy(
                table_hbm.at[idx_v.at[j]], rows_v.at[pl.ds(j * _BBLK, _BBLK), :], sem
            )
            for j in range(_TC)
        ]
        for cp in copies:
            cp.wait()

        for j in range(_TC):
            for d in range(_D):
                dt, ds = d // 8, d % 8
                p = plsc.load_gather(
                    pos_v, [jnp.zeros((16,), jnp.int32) + ((t0 + j) * _D + d)]
                )
                dvec = jnp.full((16,), d, jnp.int32)
                for b0 in range(_BBLK // 16):
                    src = plsc.load_gather(
                        rows_v, [lanes + (j * _BBLK + b0 * 16), dvec]
                    )
                    out_v[j, dt, pl.ds(ds * _BBLK + b0 * 16, 16)] = (
                        src * _SCALE + p
                    )

        pltpu.sync_copy(out_v, out_hbm.at[pl.ds(t0, _TC), :, wid, :])
        return carry

    lax.fori_loop(0, _NCHUNK, chunk_body, 0)


def kernel(x, seq_vectors):
    idx_t_major = x.T
    pos = jnp.asarray(_POS).reshape(-1)
    out4 = _sc_embed(idx_t_major, seq_vectors, pos)
    out5 = out4.reshape(MAX_SEQ_LENGTH, 4, _NW, 8, _BBLK)
    return out5.transpose(2, 4, 0, 1, 3).reshape(BATCH, MAX_SEQ_LENGTH, _D)


# speedup vs baseline: 4.4611x; 4.4611x over previous
import functools

import jax
import jax.numpy as jnp
import numpy as np
from jax import lax
from jax.experimental import pallas as pl
from jax.experimental.pallas import tpu as pltpu
from jax.experimental.pallas import tpu_sc as plsc

SEQ_VEC_SHAPE = 32
MAX_SEQ_LENGTH = 200
VOCAB_ROWS = 100000
BATCH = 4096

_NC = 2
_NS = 16
_NW = _NC * _NS
_BBLK = BATCH // _NW
_TC = 4
_NCHUNK = MAX_SEQ_LENGTH // _TC
_SCALE = float(np.sqrt(float(SEQ_VEC_SHAPE)))
_D = SEQ_VEC_SHAPE


def _pos_table() -> np.ndarray:
    initial_positions = np.arange(MAX_SEQ_LENGTH)[:, np.newaxis]
    positions = np.repeat(initial_positions, SEQ_VEC_SHAPE, axis=1)
    angle_rads = positions * (1.0 / 1000.0)
    s = np.sin(angle_rads)[::2]
    c = 1.0 - np.cos(angle_rads)[1::2]
    return np.vstack([s, c]).astype(np.float32)


_POS = _pos_table()


@functools.partial(
    pl.kernel,
    mesh=plsc.VectorSubcoreMesh(core_axis_name="c", subcore_axis_name="s"),
    out_type=jax.ShapeDtypeStruct((MAX_SEQ_LENGTH, _D // 8, _NW, 8 * _BBLK),
                                  jnp.float32),
    scratch_types=[
        pltpu.VMEM((_TC, _BBLK), jnp.int32),
        pltpu.VMEM((_TC * _BBLK, _D), jnp.float32),
        pltpu.VMEM((_TC, _D // 8, 8 * _BBLK), jnp.float32),
        pltpu.VMEM((MAX_SEQ_LENGTH * _D,), jnp.float32),
        pltpu.SemaphoreType.DMA,
    ],
    compiler_params=pltpu.CompilerParams(
        use_tc_tiling_on_sc=False, needs_layout_passes=False
    ),
)
def _sc_embed(idx_hbm, table_hbm, pos_hbm, out_hbm, idx_v, rows_v, out_v, pos_v, sem):
    wid = lax.axis_index("s") * _NC + lax.axis_index("c")
    lanes = lax.iota(jnp.int32, 16)

    pltpu.sync_copy(pos_hbm, pos_v)

    def chunk_body(c, carry):
        t0 = c * _TC
        pltpu.sync_copy(
            idx_hbm.at[pl.ds(t0, _TC), pl.ds(wid * _BBLK, _BBLK)], idx_v
        )
        copies = [
            pltpu.async_copy(
                table_hbm.at[idx_v.at[j]], rows_v.at[pl.ds(j * _BBLK, _BBLK), :], sem
            )
            for j in range(_TC)
        ]
        for cp in copies:
            cp.wait()

        @functools.partial(plsc.parallel_loop, 0, _D, unroll=4)
        def d_body(d):
            dt = d >> 3
            ds = d & 7
            for j in range(_TC):
                p = plsc.load_gather(
                    pos_v, [jnp.zeros((16,), jnp.int32) + ((t0 + j) * _D + d)]
                )
                for b0 in range(_BBLK // 16):
                    src = plsc.load_gather(
                        rows_v,
                        [lanes + (j * _BBLK + b0 * 16), jnp.zeros((16,), jnp.int32) + d],
                    )
                    out_v[j, dt, pl.ds(ds * _BBLK + b0 * 16, 16)] = (
                        src * _SCALE + p
                    )

        pltpu.sync_copy(out_v, out_hbm.at[pl.ds(t0, _TC), :, wid, :])
        return carry

    lax.fori_loop(0, _NCHUNK, chunk_body, 0)


def kernel(x, seq_vectors):
    idx_t_major = x.T
    pos = jnp.asarray(_POS).reshape(-1)
    out4 = _sc_embed(idx_t_major, seq_vectors, pos)
    out5 = out4.reshape(MAX_SEQ_LENGTH, 4, _NW, 8, _BBLK)
    return out5.transpose(2, 4, 0, 1, 3).reshape(BATCH, MAX_SEQ_LENGTH, _D)
